# baseline (speedup 1.0000x reference)
import functools

import jax
import jax.numpy as jnp
from jax.experimental import pallas as pl
from jax.experimental.pallas import tpu as pltpu

_NQ = 2048
_C = 128
_NB = 50000
_TA = 1000
_TB = 2000
_K = 9
_HW = 1024
_NCAND = 8


def _min_kernel(mb1_ref, mb2_ref, qtb_ref, qn_ref, o_ref, acc_ref):
    j = pl.program_id(0)

    def part(ref):
        mb = ref[...]
        mn = jnp.sum(mb * mb, axis=1, keepdims=True)
        aug = jnp.concatenate(
            [mb.astype(jnp.bfloat16), (-mn).astype(jnp.bfloat16),
             jnp.zeros((_TA, 7), jnp.bfloat16)], axis=1)
        return jnp.dot(aug, qtb_ref[...],
                       preferred_element_type=jnp.float32)

    s = jnp.maximum(part(mb1_ref), part(mb2_ref))
    m = jnp.max(s, axis=0, keepdims=True)

    @pl.when(j == 0)
    def _():
        acc_ref[...] = m

    @pl.when(j > 0)
    def _():
        acc_ref[...] = jnp.maximum(acc_ref[...], m)

    @pl.when(j == pl.num_programs(0) - 1)
    def _():
        o_ref[...] = jnp.sqrt(jnp.maximum(qn_ref[...] - acc_ref[...], 0.0))


def _topk_kernel(mb1_ref, mb2_ref, mb3_ref, mb4_ref, q_ref, pix_ref, o_ref,
                 top_ref):
    j = pl.program_id(0)

    @pl.when(j == 0)
    def _():
        top_ref[...] = jnp.full((16, 128), jnp.inf, jnp.float32)

    rows = []
    for img in range(2):
        p = pix_ref[0:1, img * _HW:(img + 1) * _HW]
        lan = jax.lax.broadcasted_iota(jnp.int32, (1, _HW), 1)
        for _ in range(_NCAND):
            a = jnp.argmax(p)
            rows.append(img * _HW + a)
            p = jnp.where(lan == a, -jnp.inf, p)
    qs = jnp.concatenate([q_ref[pl.ds(r, 1), :] for r in rows],
                         axis=0)

    dims = (((1,), (1,)), ((), ()))
    ones = jnp.ones((16, _C), jnp.float32)
    nt = _NB // _TB
    parts = []
    for c, ref in enumerate((mb1_ref, mb2_ref, mb3_ref, mb4_ref)):
        s = jax.lax.dot_general(qs, ref[...], dims,
                                preferred_element_type=jnp.float32)
        mnt = jax.lax.dot_general(ones, ref[...] * ref[...], dims,
                                  preferred_element_type=jnp.float32)
        d = mnt - s
        if c > 0:
            d = jnp.where(4 * j + c <= nt - 1, d, jnp.inf)
        parts.append(d)

    cand = jnp.concatenate([top_ref[...]] + parts, axis=1)
    lanes = jax.lax.broadcasted_iota(jnp.int32, cand.shape, 1)
    out_lane = lanes[:, 0:128]
    newtop = jnp.full((16, 128), jnp.inf, jnp.float32)
    for k in range(_K):
        mv = jnp.min(cand, axis=1, keepdims=True)
        am = jnp.argmin(cand, axis=1)
        cand = jnp.where(lanes == am[:, None], jnp.inf, cand)
        newtop = jnp.where(out_lane == k, mv, newtop)
    top_ref[...] = newtop

    @pl.when(j == pl.num_programs(0) - 1)
    def _():
        qn = 0.25 * jnp.sum(qs * qs, axis=1, keepdims=True)
        t9 = jnp.maximum(top_ref[...] + qn, 0.0)
        sa = jnp.sqrt(t9)
        sc = sa[:, 0:1]
        valid = out_lane < _K
        mx = jnp.max(jnp.where(valid, sa, -jnp.inf), axis=1, keepdims=True)
        e = jnp.where(valid, jnp.exp(sa - mx), 0.0)
        ssum = jnp.sum(e, axis=1, keepdims=True)
        sm0 = e[:, 0:1] / ssum
        imgv = sc * (1.0 - sm0)

        sub = jax.lax.broadcasted_iota(jnp.int32, (16, 1), 0)
        idxv = jnp.zeros((16, 1), jnp.float32)
        for i, r in enumerate(rows):
            idxv = jnp.where(sub == i, r.astype(jnp.float32), idxv)
        outs = []
        for img in range(2):
            in_img = (sub >= img * _NCAND) & (sub < (img + 1) * _NCAND)
            mxsc = jnp.max(jnp.where(in_img, sc, -jnp.inf), axis=0,
                           keepdims=True)
            elig = in_img & (sc == mxsc)
            pick = jnp.min(jnp.where(elig, idxv, 4096.0), axis=0,
                           keepdims=True)
            sel = elig & (idxv == pick)
            val = jnp.sum(jnp.where(sel, imgv, 0.0), axis=0,
                          keepdims=True)
            outs.append(jnp.broadcast_to(val, (1, 128)))
        o_ref[...] = jnp.concatenate(
            outs + [jnp.zeros((6, 128), jnp.float32)], axis=0)


@functools.partial(jax.jit, static_argnames=())
def kernel(feature_batch, memory_bank):
    B, H, W, C = feature_batch.shape
    fv2 = 2.0 * feature_batch.reshape(B * H * W, C)
    qt2 = fv2.T
    qtb = jnp.concatenate(
        [qt2.astype(jnp.bfloat16), jnp.ones((1, _NQ), jnp.bfloat16),
         jnp.zeros((7, _NQ), jnp.bfloat16)], axis=0)
    qn = (0.25 * jnp.sum(qt2 * qt2, axis=0))[None, :]

    pix = pl.pallas_call(
        _min_kernel,
        grid=(_NB // _TA // 2,),
        in_specs=[
            pl.BlockSpec((_TA, _C), lambda j: (2 * j, 0)),
            pl.BlockSpec((_TA, _C), lambda j: (2 * j + 1, 0)),
            pl.BlockSpec((136, _NQ), lambda j: (0, 0)),
            pl.BlockSpec((1, _NQ), lambda j: (0, 0)),
        ],
        out_specs=pl.BlockSpec((1, _NQ), lambda j: (0, 0)),
        out_shape=jax.ShapeDtypeStruct((1, _NQ), jnp.float32),
        scratch_shapes=[pltpu.VMEM((1, _NQ), jnp.float32)],
    )(memory_bank, memory_bank, qtb, qn)

    pixel_scores = pix.reshape(B, 1, H, W)

    nb_tiles = _NB // _TB
    img8 = pl.pallas_call(
        _topk_kernel,
        grid=((nb_tiles + 3) // 4,),
        in_specs=[
            pl.BlockSpec((_TB, _C), lambda j: (4 * j, 0)),
            pl.BlockSpec((_TB, _C),
                         lambda j: (jnp.minimum(4 * j + 1, nb_tiles - 1), 0)),
            pl.BlockSpec((_TB, _C),
                         lambda j: (jnp.minimum(4 * j + 2, nb_tiles - 1), 0)),
            pl.BlockSpec((_TB, _C),
                         lambda j: (jnp.minimum(4 * j + 3, nb_tiles - 1), 0)),
            pl.BlockSpec((_NQ, _C), lambda j: (0, 0)),
            pl.BlockSpec((1, _NQ), lambda j: (0, 0)),
        ],
        out_specs=pl.BlockSpec((8, 128), lambda j: (0, 0)),
        out_shape=jax.ShapeDtypeStruct((8, 128), jnp.float32),
        scratch_shapes=[pltpu.VMEM((16, 128), jnp.float32)],
    )(memory_bank, memory_bank, memory_bank, memory_bank, fv2, pix)

    image_scores = img8[0:B, 0]
    return (pixel_scores, image_scores)


# speedup vs baseline: 1.5817x; 1.5817x over previous
import functools

import jax
import jax.numpy as jnp
from jax.experimental import pallas as pl
from jax.experimental.pallas import tpu as pltpu

_NQ = 2048
_C = 128
_NB = 50000
_TA = 1000
_K = 9
_HW = 1024
_NCLS = 128


def _scorer_kernel(mb1_ref, mb2_ref, qt_ref, qn_ref, pix_ref, img_ref,
                   acc_ref):
    j = pl.program_id(0)

    def dpart(ref):
        mb = ref[...]
        s = jnp.dot(mb, qt_ref[...], preferred_element_type=jnp.float32)
        mn = jnp.sum(mb * mb, axis=1, keepdims=True)
        return mn - s

    d = jnp.minimum(dpart(mb1_ref), dpart(mb2_ref))
    c = d[0:_NCLS, :]
    for t in range(1, _TA // _NCLS):
        c = jnp.minimum(c, d[t * _NCLS:(t + 1) * _NCLS, :])
    rem = _TA % _NCLS
    cpart = jnp.minimum(c[0:rem, :], d[_TA - rem:_TA, :])
    c = jnp.concatenate([cpart, c[rem:_NCLS, :]], axis=0)

    @pl.when(j == 0)
    def _():
        acc_ref[...] = c

    @pl.when(j > 0)
    def _():
        acc_ref[...] = jnp.minimum(acc_ref[...], c)

    @pl.when(j == pl.num_programs(0) - 1)
    def _():
        acc = acc_ref[...]
        part = jnp.min(acc, axis=0, keepdims=True)
        pixv = jnp.sqrt(jnp.maximum(part + qn_ref[...], 0.0))
        pix_ref[...] = pixv

        lane2k = jax.lax.broadcasted_iota(jnp.int32, (1, _NQ), 1)
        sub = jax.lax.broadcasted_iota(jnp.int32, (_NCLS, 1), 0)
        l128 = jax.lax.broadcasted_iota(jnp.int32, (1, 128), 1)
        valid = l128 < _K
        outs = []
        for img in range(2):
            a = jnp.argmax(pixv[0:1, img * _HW:(img + 1) * _HW])
            g = img * _HW + a
            colmask = lane2k == g
            col = jnp.min(jnp.where(colmask, acc, jnp.inf), axis=1,
                          keepdims=True)
            qng = jnp.min(jnp.where(colmask, qn_ref[...], jnp.inf))
            t9 = jnp.full((1, 128), jnp.inf, jnp.float32)
            cur = col
            for k in range(_K):
                mv = jnp.min(cur)
                amk = jnp.argmin(cur)
                cur = jnp.where(sub == amk, jnp.inf, cur)
                t9 = jnp.where(l128 == k, mv + qng, t9)
            sa = jnp.sqrt(jnp.maximum(jnp.where(valid, t9, 0.0), 0.0))
            mx = jnp.max(jnp.where(valid, sa, -jnp.inf))
            e = jnp.where(valid, jnp.exp(sa - mx), 0.0)
            sm0 = e[0:1, 0:1] / jnp.sum(e, axis=1, keepdims=True)
            iv = sa[0:1, 0:1] * (1.0 - sm0)
            outs.append(jnp.broadcast_to(iv, (1, 128)))
        img_ref[...] = jnp.concatenate(
            outs + [jnp.zeros((6, 128), jnp.float32)], axis=0)


@functools.partial(jax.jit, static_argnames=())
def kernel(feature_batch, memory_bank):
    B, H, W, C = feature_batch.shape
    fv2 = 2.0 * feature_batch.reshape(B * H * W, C)
    qt2 = fv2.T
    qn = (0.25 * jnp.sum(qt2 * qt2, axis=0))[None, :]

    pix, img8 = pl.pallas_call(
        _scorer_kernel,
        grid=(_NB // _TA // 2,),
        in_specs=[
            pl.BlockSpec((_TA, _C), lambda j: (2 * j, 0)),
            pl.BlockSpec((_TA, _C), lambda j: (2 * j + 1, 0)),
            pl.BlockSpec((_C, _NQ), lambda j: (0, 0)),
            pl.BlockSpec((1, _NQ), lambda j: (0, 0)),
        ],
        out_specs=[
            pl.BlockSpec((1, _NQ), lambda j: (0, 0)),
            pl.BlockSpec((8, 128), lambda j: (0, 0)),
        ],
        out_shape=[
            jax.ShapeDtypeStruct((1, _NQ), jnp.float32),
            jax.ShapeDtypeStruct((8, 128), jnp.float32),
        ],
        scratch_shapes=[pltpu.VMEM((_NCLS, _NQ), jnp.float32)],
    )(memory_bank, memory_bank, qt2, qn)

    pixel_scores = pix.reshape(B, 1, H, W)
    image_scores = img8[0:B, 0]
    return (pixel_scores, image_scores)
